# Initial kernel scaffold; baseline (speedup 1.0000x reference)
import jax
import jax.numpy as jnp
from jax.experimental import pallas as pl


def kernel(x, edge_index, W1_src, W1_dst, att1_src, att1_dst, b1, Wl1, bl1, W2_src, W2_dst, att2_src, att2_dst, b2):
    raise NotImplementedError("write your pallas kernel here")


# speedup vs baseline: 22.6719x; 22.6719x over previous
import jax
import jax.numpy as jnp
from jax import lax
from jax.experimental import pallas as pl
from jax.experimental.pallas import tpu as pltpu
from jax.experimental.pallas import tpu_sc as plsc

N = 10000
E = 320000
F = 128
NC = 2
NS = 16
NW = NC * NS
EPT = E // NW
K = 80
NCH = EPT // K
NP = 10240
SLAB = NP // NS
GB = F // 16
RB = 2000
EPS = 1e-16

_f32 = jnp.float32


def _proj_body(x_ref, ws_ref, wd_ref, atts_ref, attd_ref,
               xs_ref, as_ref, ad_ref, ms_ref, md_ref):
    xb = x_ref[...]
    xs = jnp.dot(xb, ws_ref[...], preferred_element_type=_f32)
    xs_ref[...] = xs
    a_s = jnp.dot(xs, atts_ref[...], preferred_element_type=_f32)
    as_ref[...] = a_s
    xd = jnp.dot(xb, wd_ref[...], preferred_element_type=_f32)
    a_d = jnp.dot(xd, attd_ref[...], preferred_element_type=_f32)
    ad_ref[...] = a_d
    bs = jnp.max(a_s).reshape(1, 1)
    bd = jnp.max(a_d).reshape(1, 1)
    i = pl.program_id(0)

    @pl.when(i == 0)
    def _():
        ms_ref[...] = bs
        md_ref[...] = bd

    @pl.when(i > 0)
    def _():
        ms_ref[...] = jnp.maximum(ms_ref[...], bs)
        md_ref[...] = jnp.maximum(md_ref[...], bd)


def _project(x, w_src, w_dst, att_src, att_dst):
    return pl.pallas_call(
        _proj_body,
        grid=(N // RB,),
        in_specs=[
            pl.BlockSpec((RB, F), lambda i: (i, 0)),
            pl.BlockSpec((F, F), lambda i: (0, 0)),
            pl.BlockSpec((F, F), lambda i: (0, 0)),
            pl.BlockSpec((F, 1), lambda i: (0, 0)),
            pl.BlockSpec((F, 1), lambda i: (0, 0)),
        ],
        out_specs=[
            pl.BlockSpec((RB, F), lambda i: (i, 0)),
            pl.BlockSpec((RB, 1), lambda i: (i, 0)),
            pl.BlockSpec((RB, 1), lambda i: (i, 0)),
            pl.BlockSpec((1, 1), lambda i: (0, 0)),
            pl.BlockSpec((1, 1), lambda i: (0, 0)),
        ],
        out_shape=[
            jax.ShapeDtypeStruct((N, F), _f32),
            jax.ShapeDtypeStruct((N, 1), _f32),
            jax.ShapeDtypeStruct((N, 1), _f32),
            jax.ShapeDtypeStruct((1, 1), _f32),
            jax.ShapeDtypeStruct((1, 1), _f32),
        ],
    )(x, w_src, w_dst, att_src.reshape(F, 1), att_dst.reshape(F, 1))


def _combine_body(x_ref, a0_ref, a1_ref, z0_ref, z1_ref, b1_ref,
                  wl_ref, bl_ref, ws_ref, wd_ref, atts_ref, attd_ref,
                  xs_ref, as_ref, ad_ref, ms_ref, md_ref):
    zsum = z0_ref[...] + z1_ref[...] + EPS
    gat = (a0_ref[...] + a1_ref[...]) / zsum + b1_ref[...]
    h = gat + jnp.dot(x_ref[...], wl_ref[...], preferred_element_type=_f32)
    h = jnp.maximum(h + bl_ref[...], 0.0)
    xs = jnp.dot(h, ws_ref[...], preferred_element_type=_f32)
    xs_ref[...] = xs
    a_s = jnp.dot(xs, atts_ref[...], preferred_element_type=_f32)
    as_ref[...] = a_s
    xd = jnp.dot(h, wd_ref[...], preferred_element_type=_f32)
    a_d = jnp.dot(xd, attd_ref[...], preferred_element_type=_f32)
    ad_ref[...] = a_d
    bs = jnp.max(a_s).reshape(1, 1)
    bd = jnp.max(a_d).reshape(1, 1)
    i = pl.program_id(0)

    @pl.when(i == 0)
    def _():
        ms_ref[...] = bs
        md_ref[...] = bd

    @pl.when(i > 0)
    def _():
        ms_ref[...] = jnp.maximum(ms_ref[...], bs)
        md_ref[...] = jnp.maximum(md_ref[...], bd)


def _combine(x, acc, z, b1, wl, bl, w_src, w_dst, att_src, att_dst):
    full = lambda i: (0, 0)
    row = lambda i: (i, 0)
    return pl.pallas_call(
        _combine_body,
        grid=(N // RB,),
        in_specs=[
            pl.BlockSpec((RB, F), row),
            pl.BlockSpec((RB, F), row),
            pl.BlockSpec((RB, F), row),
            pl.BlockSpec((RB, 1), row),
            pl.BlockSpec((RB, 1), row),
            pl.BlockSpec((1, F), full),
            pl.BlockSpec((F, F), full),
            pl.BlockSpec((1, F), full),
            pl.BlockSpec((F, F), full),
            pl.BlockSpec((F, F), full),
            pl.BlockSpec((F, 1), full),
            pl.BlockSpec((F, 1), full),
        ],
        out_specs=[
            pl.BlockSpec((RB, F), row),
            pl.BlockSpec((RB, 1), row),
            pl.BlockSpec((RB, 1), row),
            pl.BlockSpec((1, 1), full),
            pl.BlockSpec((1, 1), full),
        ],
        out_shape=[
            jax.ShapeDtypeStruct((N, F), _f32),
            jax.ShapeDtypeStruct((N, 1), _f32),
            jax.ShapeDtypeStruct((N, 1), _f32),
            jax.ShapeDtypeStruct((1, 1), _f32),
            jax.ShapeDtypeStruct((1, 1), _f32),
        ],
    )(x, acc[0, :N], acc[1, :N], z[0, :N].reshape(N, 1), z[1, :N].reshape(N, 1),
      b1.reshape(1, F), wl, bl.reshape(1, F), w_src, w_dst,
      att_src.reshape(F, 1), att_dst.reshape(F, 1))


def _final_body(a0_ref, a1_ref, z0_ref, z1_ref, b2_ref, out_ref):
    zsum = z0_ref[...] + z1_ref[...] + EPS
    out_ref[...] = (a0_ref[...] + a1_ref[...]) / zsum + b2_ref[...]


def _final(acc, z, b2):
    full = lambda i: (0, 0)
    row = lambda i: (i, 0)
    return pl.pallas_call(
        _final_body,
        grid=(N // RB,),
        in_specs=[
            pl.BlockSpec((RB, F), row),
            pl.BlockSpec((RB, F), row),
            pl.BlockSpec((RB, 1), row),
            pl.BlockSpec((RB, 1), row),
            pl.BlockSpec((1, F), full),
        ],
        out_specs=pl.BlockSpec((RB, F), row),
        out_shape=jax.ShapeDtypeStruct((N, F), _f32),
    )(acc[0, :N], acc[1, :N], z[0, :N].reshape(N, 1), z[1, :N].reshape(N, 1),
      b2.reshape(1, F))


def _edge_body(as_hbm, ad_hbm, src_hbm, dst_hbm, xs_hbm, mv_hbm,
               outp_hbm, zp_hbm, pout_hbm,
               src_v, dst_v, as_v, ad_v, rows_v, p_v, mv_v,
               acc_sh, z_sh, sem):
    c = lax.axis_index("c")
    s = lax.axis_index("s")
    wid = s * NC + c
    base = s * SLAB

    pltpu.sync_copy(as_hbm, as_v)
    pltpu.sync_copy(ad_hbm, ad_v)
    pltpu.sync_copy(mv_hbm, mv_v)

    def zrow(r, _):
        for g in range(GB):
            rows_v[r, pl.ds(g * 16, 16)] = jnp.zeros((16,), _f32)
        return 0
    lax.fori_loop(0, K, zrow, 0)
    for l in range(K // 16):
        p_v[0, pl.ds(l * 16, 16)] = jnp.zeros((16,), _f32)

    for t in range(SLAB // K):
        pltpu.sync_copy(rows_v, acc_sh.at[pl.ds(base + t * K, K)])
        pltpu.sync_copy(p_v.at[0], z_sh.at[pl.ds(base + t * K, K)])
    plsc.subcore_barrier()

    mraw = mv_v[...]
    mtot = jnp.maximum(mraw, 0.2 * mraw)

    def chunk(j, _):
        pltpu.sync_copy(src_hbm.at[wid, j], src_v.at[0])
        pltpu.sync_copy(dst_hbm.at[wid, j], dst_v.at[0])
        pltpu.async_copy(xs_hbm.at[src_v.at[0]], rows_v, sem).wait()
        for l in range(K // 16):
            sv = src_v[0, pl.ds(l * 16, 16)]
            dv = dst_v[0, pl.ds(l * 16, 16)]
            e = plsc.load_gather(as_v, [sv]) + plsc.load_gather(ad_v, [dv])
            e = jnp.maximum(e, 0.2 * e)
            p_v[0, pl.ds(l * 16, 16)] = jnp.exp(e - mtot)

        def srow(l, _):
            pv16 = p_v[0, pl.ds(l * 16, 16)]
            r0 = l * 16
            for i in range(16):
                pr = pv16[i]
                for g in range(GB):
                    rows_v[r0 + i, pl.ds(g * 16, 16)] = (
                        rows_v[r0 + i, pl.ds(g * 16, 16)] * pr)
            return 0
        lax.fori_loop(0, K // 16, srow, 0)

        pltpu.sync_copy(rows_v, acc_sh.at[dst_v.at[0]], add=True)
        pltpu.sync_copy(p_v.at[0], z_sh.at[dst_v.at[0]], add=True)
        pltpu.sync_copy(p_v.at[0], pout_hbm.at[wid, j])
        return 0
    lax.fori_loop(0, NCH, chunk, 0)

    plsc.subcore_barrier()
    pltpu.sync_copy(acc_sh.at[pl.ds(base, SLAB)], outp_hbm.at[c, pl.ds(base, SLAB)])
    pltpu.sync_copy(z_sh.at[pl.ds(base, SLAB)], zp_hbm.at[c, pl.ds(base, SLAB)])


_edge_kernel = pl.kernel(
    _edge_body,
    out_type=(
        jax.ShapeDtypeStruct((NC, NP, F), _f32),
        jax.ShapeDtypeStruct((NC, NP), _f32),
        jax.ShapeDtypeStruct((NW, NCH, K), _f32),
    ),
    mesh=plsc.VectorSubcoreMesh(core_axis_name="c", subcore_axis_name="s",
                                num_cores=NC, num_subcores=NS),
    compiler_params=pltpu.CompilerParams(needs_layout_passes=False,
                                         internal_scratch_in_bytes=128 * 1024),
    scratch_types=[
        pltpu.VMEM((2, K), jnp.int32),
        pltpu.VMEM((2, K), jnp.int32),
        pltpu.VMEM((N,), _f32),
        pltpu.VMEM((N,), _f32),
        pltpu.VMEM((K, F), _f32),
        pltpu.VMEM((2, K), _f32),
        pltpu.VMEM((16,), _f32),
        pltpu.VMEM_SHARED((NP, F), _f32),
        pltpu.VMEM_SHARED((NP,), _f32),
        pltpu.SemaphoreType.DMA,
    ],
)


def _alpha_body(zp_hbm, p_hbm, dst_hbm, aout_hbm,
                z0_v, z1_v, p_v, dst_v, a_v):
    c = lax.axis_index("c")
    s = lax.axis_index("s")
    wid = s * NC + c
    pltpu.sync_copy(zp_hbm.at[0], z0_v)
    pltpu.sync_copy(zp_hbm.at[1], z1_v)
    pltpu.sync_copy(p_hbm.at[wid], p_v)
    pltpu.sync_copy(dst_hbm.at[wid], dst_v)

    def chunk(j, _):
        for l in range(K // 16):
            dv = dst_v[j, pl.ds(l * 16, 16)]
            z = plsc.load_gather(z0_v, [dv]) + plsc.load_gather(z1_v, [dv]) + EPS
            a_v[j, pl.ds(l * 16, 16)] = p_v[j, pl.ds(l * 16, 16)] / z
        return 0
    lax.fori_loop(0, NCH, chunk, 0)
    pltpu.sync_copy(a_v, aout_hbm.at[wid])


_alpha_kernel = pl.kernel(
    _alpha_body,
    out_type=jax.ShapeDtypeStruct((NW, NCH, K), _f32),
    mesh=plsc.VectorSubcoreMesh(core_axis_name="c", subcore_axis_name="s",
                                num_cores=NC, num_subcores=NS),
    compiler_params=pltpu.CompilerParams(needs_layout_passes=False),
    scratch_types=[
        pltpu.VMEM((NP,), _f32),
        pltpu.VMEM((NP,), _f32),
        pltpu.VMEM((NCH, K), _f32),
        pltpu.VMEM((NCH, K), jnp.int32),
        pltpu.VMEM((NCH, K), _f32),
    ],
)


def kernel(x, edge_index, W1_src, W1_dst, att1_src, att1_dst, b1, Wl1, bl1,
           W2_src, W2_dst, att2_src, att2_dst, b2):
    src3 = edge_index[0].astype(jnp.int32).reshape(NW, NCH, K)
    dst3 = edge_index[1].astype(jnp.int32).reshape(NW, NCH, K)

    xs1, as1, ad1, ms1, md1 = _project(x, W1_src, W1_dst, att1_src, att1_dst)
    mv1 = jnp.broadcast_to((ms1 + md1).reshape(()), (16,))
    acc1, z1, _ = _edge_kernel(as1.reshape(N), ad1.reshape(N), src3, dst3,
                               xs1, mv1)

    xs2, as2, ad2, ms2, md2 = _combine(x, acc1, z1, b1, Wl1, bl1,
                                       W2_src, W2_dst, att2_src, att2_dst)
    mv2 = jnp.broadcast_to((ms2 + md2).reshape(()), (16,))
    acc2, z2, p2 = _edge_kernel(as2.reshape(N), ad2.reshape(N), src3, dst3,
                                xs2, mv2)

    out = _final(acc2, z2, b2)
    alpha = _alpha_kernel(z2, p2, dst3)
    return out, alpha.reshape(E, 1)


# speedup vs baseline: 36.3860x; 1.6049x over previous
import jax
import jax.numpy as jnp
from jax import lax
from jax.experimental import pallas as pl
from jax.experimental.pallas import tpu as pltpu
from jax.experimental.pallas import tpu_sc as plsc

N = 10000
E = 320000
F = 128
NC = 2
NS = 16
NW = NC * NS
EPT = E // NW
K = 80
NCH = EPT // K
IB = 5
NB = NCH // IB
NP = 10240
SLAB = NP // NS
GB = F // 16
RB = 2000
EPS = 1e-16

_f32 = jnp.float32


def _proj_body(x_ref, ws_ref, wd_ref, atts_ref, attd_ref,
               xs_ref, as_ref, ad_ref, ms_ref, md_ref):
    xb = x_ref[...]
    xs = jnp.dot(xb, ws_ref[...], preferred_element_type=_f32)
    xs_ref[...] = xs
    a_s = jnp.dot(xs, atts_ref[...], preferred_element_type=_f32)
    as_ref[...] = a_s
    xd = jnp.dot(xb, wd_ref[...], preferred_element_type=_f32)
    a_d = jnp.dot(xd, attd_ref[...], preferred_element_type=_f32)
    ad_ref[...] = a_d
    bs = jnp.max(a_s).reshape(1, 1)
    bd = jnp.max(a_d).reshape(1, 1)
    i = pl.program_id(0)

    @pl.when(i == 0)
    def _():
        ms_ref[...] = bs
        md_ref[...] = bd

    @pl.when(i > 0)
    def _():
        ms_ref[...] = jnp.maximum(ms_ref[...], bs)
        md_ref[...] = jnp.maximum(md_ref[...], bd)


def _project(x, w_src, w_dst, att_src, att_dst):
    return pl.pallas_call(
        _proj_body,
        grid=(N // RB,),
        in_specs=[
            pl.BlockSpec((RB, F), lambda i: (i, 0)),
            pl.BlockSpec((F, F), lambda i: (0, 0)),
            pl.BlockSpec((F, F), lambda i: (0, 0)),
            pl.BlockSpec((F, 1), lambda i: (0, 0)),
            pl.BlockSpec((F, 1), lambda i: (0, 0)),
        ],
        out_specs=[
            pl.BlockSpec((RB, F), lambda i: (i, 0)),
            pl.BlockSpec((RB, 1), lambda i: (i, 0)),
            pl.BlockSpec((RB, 1), lambda i: (i, 0)),
            pl.BlockSpec((1, 1), lambda i: (0, 0)),
            pl.BlockSpec((1, 1), lambda i: (0, 0)),
        ],
        out_shape=[
            jax.ShapeDtypeStruct((N, F), _f32),
            jax.ShapeDtypeStruct((N, 1), _f32),
            jax.ShapeDtypeStruct((N, 1), _f32),
            jax.ShapeDtypeStruct((1, 1), _f32),
            jax.ShapeDtypeStruct((1, 1), _f32),
        ],
    )(x, w_src, w_dst, att_src.reshape(F, 1), att_dst.reshape(F, 1))


def _combine_body(x_ref, a0_ref, a1_ref, z0_ref, z1_ref, b1_ref,
                  wl_ref, bl_ref, ws_ref, wd_ref, atts_ref, attd_ref,
                  xs_ref, as_ref, ad_ref, ms_ref, md_ref):
    zsum = z0_ref[...] + z1_ref[...] + EPS
    gat = (a0_ref[...] + a1_ref[...]) / zsum + b1_ref[...]
    h = gat + jnp.dot(x_ref[...], wl_ref[...], preferred_element_type=_f32)
    h = jnp.maximum(h + bl_ref[...], 0.0)
    xs = jnp.dot(h, ws_ref[...], preferred_element_type=_f32)
    xs_ref[...] = xs
    a_s = jnp.dot(xs, atts_ref[...], preferred_element_type=_f32)
    as_ref[...] = a_s
    xd = jnp.dot(h, wd_ref[...], preferred_element_type=_f32)
    a_d = jnp.dot(xd, attd_ref[...], preferred_element_type=_f32)
    ad_ref[...] = a_d
    bs = jnp.max(a_s).reshape(1, 1)
    bd = jnp.max(a_d).reshape(1, 1)
    i = pl.program_id(0)

    @pl.when(i == 0)
    def _():
        ms_ref[...] = bs
        md_ref[...] = bd

    @pl.when(i > 0)
    def _():
        ms_ref[...] = jnp.maximum(ms_ref[...], bs)
        md_ref[...] = jnp.maximum(md_ref[...], bd)


def _combine(x, acc, z, b1, wl, bl, w_src, w_dst, att_src, att_dst):
    full = lambda i: (0, 0)
    row = lambda i: (i, 0)
    return pl.pallas_call(
        _combine_body,
        grid=(N // RB,),
        in_specs=[
            pl.BlockSpec((RB, F), row),
            pl.BlockSpec((RB, F), row),
            pl.BlockSpec((RB, F), row),
            pl.BlockSpec((RB, 1), row),
            pl.BlockSpec((RB, 1), row),
            pl.BlockSpec((1, F), full),
            pl.BlockSpec((F, F), full),
            pl.BlockSpec((1, F), full),
            pl.BlockSpec((F, F), full),
            pl.BlockSpec((F, F), full),
            pl.BlockSpec((F, 1), full),
            pl.BlockSpec((F, 1), full),
        ],
        out_specs=[
            pl.BlockSpec((RB, F), row),
            pl.BlockSpec((RB, 1), row),
            pl.BlockSpec((RB, 1), row),
            pl.BlockSpec((1, 1), full),
            pl.BlockSpec((1, 1), full),
        ],
        out_shape=[
            jax.ShapeDtypeStruct((N, F), _f32),
            jax.ShapeDtypeStruct((N, 1), _f32),
            jax.ShapeDtypeStruct((N, 1), _f32),
            jax.ShapeDtypeStruct((1, 1), _f32),
            jax.ShapeDtypeStruct((1, 1), _f32),
        ],
    )(x, acc[0, :N], acc[1, :N], z[0, :N].reshape(N, 1), z[1, :N].reshape(N, 1),
      b1.reshape(1, F), wl, bl.reshape(1, F), w_src, w_dst,
      att_src.reshape(F, 1), att_dst.reshape(F, 1))


def _final_body(a0_ref, a1_ref, z0_ref, z1_ref, b2_ref, out_ref):
    zsum = z0_ref[...] + z1_ref[...] + EPS
    out_ref[...] = (a0_ref[...] + a1_ref[...]) / zsum + b2_ref[...]


def _final(acc, z, b2):
    full = lambda i: (0, 0)
    row = lambda i: (i, 0)
    return pl.pallas_call(
        _final_body,
        grid=(N // RB,),
        in_specs=[
            pl.BlockSpec((RB, F), row),
            pl.BlockSpec((RB, F), row),
            pl.BlockSpec((RB, 1), row),
            pl.BlockSpec((RB, 1), row),
            pl.BlockSpec((1, F), full),
        ],
        out_specs=pl.BlockSpec((RB, F), row),
        out_shape=jax.ShapeDtypeStruct((N, F), _f32),
    )(acc[0, :N], acc[1, :N], z[0, :N].reshape(N, 1), z[1, :N].reshape(N, 1),
      b2.reshape(1, F))


def _edge_body(as_hbm, ad_hbm, src_hbm, dst_hbm, xs_hbm, mv_hbm,
               outp_hbm, zp_hbm, pout_hbm,
               sidx_v, didx_v, as_v, ad_v, rows_v, p_v, zsrc_v, mv_v,
               acc_sh, z_sh, sem_ix, sem_g, sem_rs, sem_zs, sem_po):
    c = lax.axis_index("c")
    s = lax.axis_index("s")
    wid = s * NC + c
    base = s * SLAB

    pltpu.sync_copy(as_hbm, as_v)
    pltpu.sync_copy(ad_hbm, ad_v)
    pltpu.sync_copy(mv_hbm, mv_v)

    def zrow(r, _):
        for g in range(GB):
            rows_v[r, pl.ds(g * 16, 16)] = jnp.zeros((16,), _f32)
        return 0
    lax.fori_loop(0, K, zrow, 0)
    for l in range(K // 16):
        zsrc_v[0, pl.ds(l * 16, 16)] = jnp.zeros((16,), _f32)

    for t in range(SLAB // K):
        pltpu.sync_copy(rows_v.at[pl.ds(0, K)],
                        acc_sh.at[pl.ds(base + t * K, K)])
        pltpu.sync_copy(zsrc_v.at[0], z_sh.at[pl.ds(base + t * K, K)])
    plsc.subcore_barrier()

    mraw = mv_v[...]
    mtot = jnp.maximum(mraw, 0.2 * mraw)

    def ix_descs(j, q):
        return (pltpu.make_async_copy(src_hbm.at[wid, j],
                                      sidx_v.at[q], sem_ix.at[q]),
                pltpu.make_async_copy(dst_hbm.at[wid, j],
                                      didx_v.at[q], sem_ix.at[q]))

    def g_desc(q, r):
        return pltpu.make_async_copy(xs_hbm.at[sidx_v.at[q]],
                                     rows_v.at[pl.ds(r * K, K)], sem_g.at[r])

    def rs_desc(q, r):
        return pltpu.make_async_copy(rows_v.at[pl.ds(r * K, K)],
                                     acc_sh.at[didx_v.at[q]],
                                     sem_rs.at[r])

    def zs_desc(q, r):
        return pltpu.make_async_copy(zsrc_v.at[r],
                                     z_sh.at[didx_v.at[q]],
                                     sem_zs.at[r])

    def po_desc(b, rb):
        return pltpu.make_async_copy(p_v.at[pl.ds(rb * IB, IB)],
                                     pout_hbm.at[wid, b], sem_po.at[rb])

    d0, d1 = ix_descs(0, 0)
    d0.start()
    d1.start()
    e0, e1 = ix_descs(1, 1)
    e0.start()
    e1.start()
    d0.wait()
    d1.wait()
    g_desc(0, 0).start()

    def chunk(j, _):
        b = lax.div(j, IB)
        j0 = lax.rem(j, IB)
        q = lax.rem(j, 4)
        qn = lax.rem(j + 1, 4)
        q2 = lax.rem(j + 2, 4)
        r = lax.rem(j, 2)
        rn = lax.rem(j + 1, 2)
        rb = lax.rem(b, 2)

        g_desc(q, r).wait()

        @pl.when(j >= 2)
        def _():
            zs_desc(q2, r).wait()

        @pl.when(j + 2 < NCH)
        def _():
            f0, f1 = ix_descs(j + 2, q2)
            f0.start()
            f1.start()

        @pl.when((j0 == 0) & (b > 1))
        def _():
            po_desc(b - 2, rb).wait()

        prow = rb * IB + j0
        for l in range(K // 16):
            sv = sidx_v[q, pl.ds(l * 16, 16)]
            dv = didx_v[q, pl.ds(l * 16, 16)]
            e = plsc.load_gather(as_v, [sv]) + plsc.load_gather(ad_v, [dv])
            e = jnp.maximum(e, 0.2 * e)
            pv = jnp.exp(e - mtot)
            pv = pv
            zsrc_v[r, pl.ds(l * 16, 16)] = pv
            p_v[prow, pl.ds(l * 16, 16)] = pv

        rbase = r * K
        def srow(l, _):
            pv16 = zsrc_v[r, pl.ds(l * 16, 16)]
            r0 = l * 16
            for i in range(16):
                pr = pv16[i]
                for g in range(GB):
                    rows_v[rbase + r0 + i, pl.ds(g * 16, 16)] = (
                        rows_v[rbase + r0 + i, pl.ds(g * 16, 16)] * pr)
            return 0
        lax.fori_loop(0, K // 16, srow, 0)

        @pl.when(j + 1 < NCH)
        def _():
            @pl.when(j >= 1)
            def _():
                rs_desc(lax.rem(j - 1, 4), rn).wait()
            f0, f1 = ix_descs(j + 1, qn)
            f0.wait()
            f1.wait()
            g_desc(qn, rn).start()

        pltpu.async_copy(rows_v.at[pl.ds(r * K, K)], acc_sh.at[didx_v.at[q]],
                         sem_rs.at[r], add=True)
        pltpu.async_copy(zsrc_v.at[r], z_sh.at[didx_v.at[q]],
                         sem_zs.at[r], add=True)

        @pl.when(j0 == IB - 1)
        def _():
            po_desc(b, rb).start()
        return 0
    lax.fori_loop(0, NCH, chunk, 0)

    for jj in (NCH - 2, NCH - 1):
        r = jj % 2
        q = jj % 4
        rs_desc(q, r).wait()
        zs_desc(q, r).wait()
    po_desc(NB - 2, (NB - 2) % 2).wait()
    po_desc(NB - 1, (NB - 1) % 2).wait()

    plsc.subcore_barrier()
    pltpu.sync_copy(acc_sh.at[pl.ds(base, SLAB)], outp_hbm.at[c, pl.ds(base, SLAB)])
    pltpu.sync_copy(z_sh.at[pl.ds(base, SLAB)], zp_hbm.at[c, pl.ds(base, SLAB)])


_edge_kernel = pl.kernel(
    _edge_body,
    out_type=(
        jax.ShapeDtypeStruct((NC, NP, F), _f32),
        jax.ShapeDtypeStruct((NC, NP), _f32),
        jax.ShapeDtypeStruct((NW, NB, IB, 128), _f32),
    ),
    mesh=plsc.VectorSubcoreMesh(core_axis_name="c", subcore_axis_name="s",
                                num_cores=NC, num_subcores=NS),
    compiler_params=pltpu.CompilerParams(needs_layout_passes=False,
                                         internal_scratch_in_bytes=128 * 1024),
    scratch_types=[
        pltpu.VMEM((4, K), jnp.int32),
        pltpu.VMEM((4, K), jnp.int32),
        pltpu.VMEM((N,), _f32),
        pltpu.VMEM((N,), _f32),
        pltpu.VMEM((2 * K, F), _f32),
        pltpu.VMEM((2 * IB, 128), _f32),
        pltpu.VMEM((2, K), _f32),
        pltpu.VMEM((16,), _f32),
        pltpu.VMEM_SHARED((NP, F), _f32),
        pltpu.VMEM_SHARED((NP,), _f32),
        pltpu.SemaphoreType.DMA((4,)),
        pltpu.SemaphoreType.DMA((2,)),
        pltpu.SemaphoreType.DMA((2,)),
        pltpu.SemaphoreType.DMA((2,)),
        pltpu.SemaphoreType.DMA((2,)),
    ],
)


def _alpha_body(zp_hbm, p_hbm, dst_hbm, aout_hbm,
                z0_v, z1_v, p_v, dst_v, a_v):
    c = lax.axis_index("c")
    s = lax.axis_index("s")
    wid = s * NC + c
    pltpu.sync_copy(zp_hbm.at[0], z0_v)
    pltpu.sync_copy(zp_hbm.at[1], z1_v)
    pltpu.sync_copy(p_hbm.at[wid], p_v)
    pltpu.sync_copy(dst_hbm.at[wid], dst_v)

    def chunk(j, _):
        for l in range(K // 16):
            dv = dst_v[j, pl.ds(l * 16, 16)]
            z = (plsc.load_gather(z0_v, [dv])
                 + plsc.load_gather(z1_v, [dv]) + EPS)
            a_v[j, pl.ds(l * 16, 16)] = p_v[j, pl.ds(l * 16, 16)] / z
        return 0
    lax.fori_loop(0, NCH, chunk, 0)
    pltpu.sync_copy(a_v, aout_hbm.at[wid])


_alpha_kernel = pl.kernel(
    _alpha_body,
    out_type=jax.ShapeDtypeStruct((NW, NCH, K), _f32),
    mesh=plsc.VectorSubcoreMesh(core_axis_name="c", subcore_axis_name="s",
                                num_cores=NC, num_subcores=NS),
    compiler_params=pltpu.CompilerParams(needs_layout_passes=False),
    scratch_types=[
        pltpu.VMEM((NP,), _f32),
        pltpu.VMEM((NP,), _f32),
        pltpu.VMEM((NCH, 128), _f32),
        pltpu.VMEM((NCH, K), jnp.int32),
        pltpu.VMEM((NCH, K), _f32),
    ],
)


def kernel(x, edge_index, W1_src, W1_dst, att1_src, att1_dst, b1, Wl1, bl1,
           W2_src, W2_dst, att2_src, att2_dst, b2):
    src3 = edge_index[0].astype(jnp.int32).reshape(NW, NCH, K)
    dst3 = edge_index[1].astype(jnp.int32).reshape(NW, NCH, K)

    xs1, as1, ad1, ms1, md1 = _project(x, W1_src, W1_dst, att1_src, att1_dst)
    mv1 = jnp.broadcast_to((ms1 + md1).reshape(()), (16,))
    acc1, z1, _ = _edge_kernel(as1.reshape(N), ad1.reshape(N), src3, dst3,
                               xs1, mv1)

    xs2, as2, ad2, ms2, md2 = _combine(x, acc1, z1, b1, Wl1, bl1,
                                       W2_src, W2_dst, att2_src, att2_dst)
    mv2 = jnp.broadcast_to((ms2 + md2).reshape(()), (16,))
    acc2, z2, p2 = _edge_kernel(as2.reshape(N), ad2.reshape(N), src3, dst3,
                                xs2, mv2)

    out = _final(acc2, z2, b2)
    alpha = _alpha_kernel(z2, p2.reshape(NW, NCH, 128), dst3)
    return out, alpha.reshape(E, 1)


# speedup vs baseline: 36.9057x; 1.0143x over previous
import jax
import jax.numpy as jnp
from jax import lax
from jax.experimental import pallas as pl
from jax.experimental.pallas import tpu as pltpu
from jax.experimental.pallas import tpu_sc as plsc

N = 10000
E = 320000
F = 128
NC = 2
NS = 16
NW = NC * NS
EPT = E // NW
K = 80
NCH = EPT // K
IB = 5
NB = NCH // IB
NP = 10240
SLAB = NP // NS
GB = F // 16
RB = 2000
EPS = 1e-16

_f32 = jnp.float32


def _proj_body(x_ref, ws_ref, wd_ref, atts_ref, attd_ref,
               xs_ref, as_ref, ad_ref, ms_ref, md_ref):
    xb = x_ref[...]
    xs = jnp.dot(xb, ws_ref[...], preferred_element_type=_f32)
    xs_ref[...] = xs
    a_s = jnp.dot(xs, atts_ref[...], preferred_element_type=_f32)
    as_ref[...] = a_s
    xd = jnp.dot(xb, wd_ref[...], preferred_element_type=_f32)
    a_d = jnp.dot(xd, attd_ref[...], preferred_element_type=_f32)
    ad_ref[...] = a_d
    bs = jnp.max(a_s).reshape(1, 1)
    bd = jnp.max(a_d).reshape(1, 1)
    i = pl.program_id(0)

    @pl.when(i == 0)
    def _():
        ms_ref[...] = bs
        md_ref[...] = bd

    @pl.when(i > 0)
    def _():
        ms_ref[...] = jnp.maximum(ms_ref[...], bs)
        md_ref[...] = jnp.maximum(md_ref[...], bd)


def _project(x, w_src, w_dst, att_src, att_dst):
    return pl.pallas_call(
        _proj_body,
        grid=(N // RB,),
        in_specs=[
            pl.BlockSpec((RB, F), lambda i: (i, 0)),
            pl.BlockSpec((F, F), lambda i: (0, 0)),
            pl.BlockSpec((F, F), lambda i: (0, 0)),
            pl.BlockSpec((F, 1), lambda i: (0, 0)),
            pl.BlockSpec((F, 1), lambda i: (0, 0)),
        ],
        out_specs=[
            pl.BlockSpec((RB, F), lambda i: (i, 0)),
            pl.BlockSpec((RB, 1), lambda i: (i, 0)),
            pl.BlockSpec((RB, 1), lambda i: (i, 0)),
            pl.BlockSpec((1, 1), lambda i: (0, 0)),
            pl.BlockSpec((1, 1), lambda i: (0, 0)),
        ],
        out_shape=[
            jax.ShapeDtypeStruct((N, F), _f32),
            jax.ShapeDtypeStruct((N, 1), _f32),
            jax.ShapeDtypeStruct((N, 1), _f32),
            jax.ShapeDtypeStruct((1, 1), _f32),
            jax.ShapeDtypeStruct((1, 1), _f32),
        ],
    )(x, w_src, w_dst, att_src.reshape(F, 1), att_dst.reshape(F, 1))


def _combine_body(x_ref, a0_ref, a1_ref, z0_ref, z1_ref, b1_ref,
                  wl_ref, bl_ref, ws_ref, wd_ref, atts_ref, attd_ref,
                  xs_ref, as_ref, ad_ref, ms_ref, md_ref):
    zsum = z0_ref[0] + z1_ref[0] + EPS
    gat = (a0_ref[0] + a1_ref[0]) / zsum + b1_ref[...]
    h = gat + jnp.dot(x_ref[...], wl_ref[...], preferred_element_type=_f32)
    h = jnp.maximum(h + bl_ref[...], 0.0)
    xs = jnp.dot(h, ws_ref[...], preferred_element_type=_f32)
    xs_ref[...] = xs
    a_s = jnp.dot(xs, atts_ref[...], preferred_element_type=_f32)
    as_ref[...] = a_s
    xd = jnp.dot(h, wd_ref[...], preferred_element_type=_f32)
    a_d = jnp.dot(xd, attd_ref[...], preferred_element_type=_f32)
    ad_ref[...] = a_d
    bs = jnp.max(a_s).reshape(1, 1)
    bd = jnp.max(a_d).reshape(1, 1)
    i = pl.program_id(0)

    @pl.when(i == 0)
    def _():
        ms_ref[...] = bs
        md_ref[...] = bd

    @pl.when(i > 0)
    def _():
        ms_ref[...] = jnp.maximum(ms_ref[...], bs)
        md_ref[...] = jnp.maximum(md_ref[...], bd)


def _combine(x, acc, z, b1, wl, bl, w_src, w_dst, att_src, att_dst):
    full = lambda i: (0, 0)
    row = lambda i: (i, 0)
    return pl.pallas_call(
        _combine_body,
        grid=(N // RB,),
        in_specs=[
            pl.BlockSpec((RB, F), row),
            pl.BlockSpec((1, RB, F), lambda i: (0, i, 0)),
            pl.BlockSpec((1, RB, F), lambda i: (1, i, 0)),
            pl.BlockSpec((1, RB, 1), lambda i: (0, i, 0)),
            pl.BlockSpec((1, RB, 1), lambda i: (1, i, 0)),
            pl.BlockSpec((1, F), full),
            pl.BlockSpec((F, F), full),
            pl.BlockSpec((1, F), full),
            pl.BlockSpec((F, F), full),
            pl.BlockSpec((F, F), full),
            pl.BlockSpec((F, 1), full),
            pl.BlockSpec((F, 1), full),
        ],
        out_specs=[
            pl.BlockSpec((RB, F), row),
            pl.BlockSpec((RB, 1), row),
            pl.BlockSpec((RB, 1), row),
            pl.BlockSpec((1, 1), full),
            pl.BlockSpec((1, 1), full),
        ],
        out_shape=[
            jax.ShapeDtypeStruct((N, F), _f32),
            jax.ShapeDtypeStruct((N, 1), _f32),
            jax.ShapeDtypeStruct((N, 1), _f32),
            jax.ShapeDtypeStruct((1, 1), _f32),
            jax.ShapeDtypeStruct((1, 1), _f32),
        ],
    )(x, acc, acc, z.reshape(NC, NP, 1), z.reshape(NC, NP, 1),
      b1.reshape(1, F), wl, bl.reshape(1, F), w_src, w_dst,
      att_src.reshape(F, 1), att_dst.reshape(F, 1))


def _final_body(a0_ref, a1_ref, z0_ref, z1_ref, b2_ref, out_ref):
    zsum = z0_ref[0] + z1_ref[0] + EPS
    out_ref[...] = (a0_ref[0] + a1_ref[0]) / zsum + b2_ref[...]


def _final(acc, z, b2):
    full = lambda i: (0, 0)
    row = lambda i: (i, 0)
    return pl.pallas_call(
        _final_body,
        grid=(N // RB,),
        in_specs=[
            pl.BlockSpec((1, RB, F), lambda i: (0, i, 0)),
            pl.BlockSpec((1, RB, F), lambda i: (1, i, 0)),
            pl.BlockSpec((1, RB, 1), lambda i: (0, i, 0)),
            pl.BlockSpec((1, RB, 1), lambda i: (1, i, 0)),
            pl.BlockSpec((1, F), full),
        ],
        out_specs=pl.BlockSpec((RB, F), row),
        out_shape=jax.ShapeDtypeStruct((N, F), _f32),
    )(acc, acc, z.reshape(NC, NP, 1), z.reshape(NC, NP, 1), b2.reshape(1, F))


def _edge_body(as_hbm, ad_hbm, src_hbm, dst_hbm, xs_hbm, mv_hbm,
               outp_hbm, zp_hbm, pout_hbm,
               sidx_v, didx_v, as_v, ad_v, rows_v, p_v, zsrc_v, mv_v,
               acc_sh, z_sh, sem_ix, sem_g, sem_rs, sem_zs, sem_po):
    c = lax.axis_index("c")
    s = lax.axis_index("s")
    wid = s * NC + c
    base = s * SLAB

    pltpu.sync_copy(as_hbm, as_v)
    pltpu.sync_copy(ad_hbm, ad_v)
    pltpu.sync_copy(mv_hbm, mv_v)

    def zrow(r, _):
        for g in range(GB):
            rows_v[r, pl.ds(g * 16, 16)] = jnp.zeros((16,), _f32)
        return 0
    lax.fori_loop(0, K, zrow, 0)
    for l in range(K // 16):
        zsrc_v[0, pl.ds(l * 16, 16)] = jnp.zeros((16,), _f32)

    for t in range(SLAB // K):
        pltpu.sync_copy(rows_v.at[pl.ds(0, K)],
                        acc_sh.at[pl.ds(base + t * K, K)])
        pltpu.sync_copy(zsrc_v.at[0], z_sh.at[pl.ds(base + t * K, K)])
    plsc.subcore_barrier()

    mraw = mv_v[...]
    mtot = jnp.maximum(mraw, 0.2 * mraw)

    def ix_descs(j, q):
        return (pltpu.make_async_copy(src_hbm.at[wid, j],
                                      sidx_v.at[q], sem_ix.at[q]),
                pltpu.make_async_copy(dst_hbm.at[wid, j],
                                      didx_v.at[q], sem_ix.at[q]))

    def g_desc(q, r):
        return pltpu.make_async_copy(xs_hbm.at[sidx_v.at[q]],
                                     rows_v.at[pl.ds(r * K, K)], sem_g.at[r])

    def rs_desc(q, r):
        return pltpu.make_async_copy(rows_v.at[pl.ds(r * K, K)],
                                     acc_sh.at[didx_v.at[q]],
                                     sem_rs.at[r])

    def zs_desc(q, r):
        return pltpu.make_async_copy(zsrc_v.at[r],
                                     z_sh.at[didx_v.at[q]],
                                     sem_zs.at[r])

    def po_desc(b, rb):
        return pltpu.make_async_copy(p_v.at[pl.ds(rb * IB, IB)],
                                     pout_hbm.at[wid, b], sem_po.at[rb])

    d0, d1 = ix_descs(0, 0)
    d0.start()
    d1.start()
    e0, e1 = ix_descs(1, 1)
    e0.start()
    e1.start()
    d0.wait()
    d1.wait()
    g_desc(0, 0).start()

    def chunk(j, _):
        b = lax.div(j, IB)
        j0 = lax.rem(j, IB)
        q = lax.rem(j, 4)
        qn = lax.rem(j + 1, 4)
        q2 = lax.rem(j + 2, 4)
        r = lax.rem(j, 2)
        rn = lax.rem(j + 1, 2)
        rb = lax.rem(b, 2)

        g_desc(q, r).wait()

        @pl.when(j >= 2)
        def _():
            zs_desc(q2, r).wait()

        @pl.when(j + 2 < NCH)
        def _():
            f0, f1 = ix_descs(j + 2, q2)
            f0.start()
            f1.start()

        @pl.when((j0 == 0) & (b > 1))
        def _():
            po_desc(b - 2, rb).wait()

        prow = rb * IB + j0
        for l in range(K // 16):
            sv = sidx_v[q, pl.ds(l * 16, 16)]
            dv = didx_v[q, pl.ds(l * 16, 16)]
            e = plsc.load_gather(as_v, [sv]) + plsc.load_gather(ad_v, [dv])
            e = jnp.maximum(e, 0.2 * e)
            pv = jnp.exp(e - mtot)
            pv = pv
            zsrc_v[r, pl.ds(l * 16, 16)] = pv
            p_v[prow, pl.ds(l * 16, 16)] = pv

        rbase = r * K
        def srow(l, _):
            pv16 = zsrc_v[r, pl.ds(l * 16, 16)]
            r0 = l * 16
            for i in range(16):
                pr = pv16[i]
                for g in range(GB):
                    rows_v[rbase + r0 + i, pl.ds(g * 16, 16)] = (
                        rows_v[rbase + r0 + i, pl.ds(g * 16, 16)] * pr)
            return 0
        lax.fori_loop(0, K // 16, srow, 0)

        @pl.when(j + 1 < NCH)
        def _():
            @pl.when(j >= 1)
            def _():
                rs_desc(lax.rem(j - 1, 4), rn).wait()
            f0, f1 = ix_descs(j + 1, qn)
            f0.wait()
            f1.wait()
            g_desc(qn, rn).start()

        pltpu.async_copy(rows_v.at[pl.ds(r * K, K)], acc_sh.at[didx_v.at[q]],
                         sem_rs.at[r], add=True)
        pltpu.async_copy(zsrc_v.at[r], z_sh.at[didx_v.at[q]],
                         sem_zs.at[r], add=True)

        @pl.when(j0 == IB - 1)
        def _():
            po_desc(b, rb).start()
        return 0
    lax.fori_loop(0, NCH, chunk, 0)

    for jj in (NCH - 2, NCH - 1):
        r = jj % 2
        q = jj % 4
        rs_desc(q, r).wait()
        zs_desc(q, r).wait()
    po_desc(NB - 2, (NB - 2) % 2).wait()
    po_desc(NB - 1, (NB - 1) % 2).wait()

    plsc.subcore_barrier()
    pltpu.sync_copy(acc_sh.at[pl.ds(base, SLAB)], outp_hbm.at[c, pl.ds(base, SLAB)])
    pltpu.sync_copy(z_sh.at[pl.ds(base, SLAB)], zp_hbm.at[c, pl.ds(base, SLAB)])


_edge_kernel = pl.kernel(
    _edge_body,
    out_type=(
        jax.ShapeDtypeStruct((NC, NP, F), _f32),
        jax.ShapeDtypeStruct((NC, NP), _f32),
        jax.ShapeDtypeStruct((NW, NB, IB, 128), _f32),
    ),
    mesh=plsc.VectorSubcoreMesh(core_axis_name="c", subcore_axis_name="s",
                                num_cores=NC, num_subcores=NS),
    compiler_params=pltpu.CompilerParams(needs_layout_passes=False,
                                         internal_scratch_in_bytes=128 * 1024),
    scratch_types=[
        pltpu.VMEM((4, K), jnp.int32),
        pltpu.VMEM((4, K), jnp.int32),
        pltpu.VMEM((N,), _f32),
        pltpu.VMEM((N,), _f32),
        pltpu.VMEM((2 * K, F), _f32),
        pltpu.VMEM((2 * IB, 128), _f32),
        pltpu.VMEM((2, K), _f32),
        pltpu.VMEM((16,), _f32),
        pltpu.VMEM_SHARED((NP, F), _f32),
        pltpu.VMEM_SHARED((NP,), _f32),
        pltpu.SemaphoreType.DMA((4,)),
        pltpu.SemaphoreType.DMA((2,)),
        pltpu.SemaphoreType.DMA((2,)),
        pltpu.SemaphoreType.DMA((2,)),
        pltpu.SemaphoreType.DMA((2,)),
    ],
)


def _alpha_body(zp_hbm, p_hbm, dst_hbm, aout_hbm,
                z0_v, z1_v, p_v, dst_v, a_v):
    c = lax.axis_index("c")
    s = lax.axis_index("s")
    wid = s * NC + c
    pltpu.sync_copy(zp_hbm.at[0], z0_v)
    pltpu.sync_copy(zp_hbm.at[1], z1_v)
    pltpu.sync_copy(p_hbm.at[wid], p_v)
    pltpu.sync_copy(dst_hbm.at[wid], dst_v)

    def chunk(j, _):
        for l in range(K // 16):
            dv = dst_v[j, pl.ds(l * 16, 16)]
            z = (plsc.load_gather(z0_v, [dv])
                 + plsc.load_gather(z1_v, [dv]) + EPS)
            a_v[j, pl.ds(l * 16, 16)] = p_v[j, pl.ds(l * 16, 16)] / z
        return 0
    lax.fori_loop(0, NCH, chunk, 0)
    pltpu.sync_copy(a_v, aout_hbm.at[wid])


_alpha_kernel = pl.kernel(
    _alpha_body,
    out_type=jax.ShapeDtypeStruct((NW, NCH, K), _f32),
    mesh=plsc.VectorSubcoreMesh(core_axis_name="c", subcore_axis_name="s",
                                num_cores=NC, num_subcores=NS),
    compiler_params=pltpu.CompilerParams(needs_layout_passes=False),
    scratch_types=[
        pltpu.VMEM((NP,), _f32),
        pltpu.VMEM((NP,), _f32),
        pltpu.VMEM((NCH, 128), _f32),
        pltpu.VMEM((NCH, K), jnp.int32),
        pltpu.VMEM((NCH, K), _f32),
    ],
)


def kernel(x, edge_index, W1_src, W1_dst, att1_src, att1_dst, b1, Wl1, bl1,
           W2_src, W2_dst, att2_src, att2_dst, b2):
    src3 = edge_index[0].astype(jnp.int32).reshape(NW, NCH, K)
    dst3 = edge_index[1].astype(jnp.int32).reshape(NW, NCH, K)

    xs1, as1, ad1, ms1, md1 = _project(x, W1_src, W1_dst, att1_src, att1_dst)
    mv1 = jnp.broadcast_to((ms1 + md1).reshape(()), (16,))
    acc1, z1, _ = _edge_kernel(as1.reshape(N), ad1.reshape(N), src3, dst3,
                               xs1, mv1)

    xs2, as2, ad2, ms2, md2 = _combine(x, acc1, z1, b1, Wl1, bl1,
                                       W2_src, W2_dst, att2_src, att2_dst)
    mv2 = jnp.broadcast_to((ms2 + md2).reshape(()), (16,))
    acc2, z2, p2 = _edge_kernel(as2.reshape(N), ad2.reshape(N), src3, dst3,
                                xs2, mv2)

    out = _final(acc2, z2, b2)
    alpha = _alpha_kernel(z2, p2.reshape(NW, NCH, 128), dst3)
    return out, alpha.reshape(E, 1)
